# baseline (speedup 1.0000x reference)
import functools

import jax
import jax.numpy as jnp
from jax import lax
from jax.experimental import pallas as pl
from jax.experimental.pallas import tpu as pltpu
from jax.experimental.pallas import tpu_sc as plsc

N_NEG = 10
K = 30
NSEL = 32
BIGI = 2**30
SEG = 2000
NSEG = 500
NCHUNK = 25
SEG_PER_CHUNK = NSEG // NCHUNK
CH = SEG * SEG_PER_CHUNK


_WIN_FULL = ((0, 4096), (4096, 4096), (8192, 4096), (12288, 3712))
_WIN_TAIL = ((0, 4096), (4096, 3840))
MPAD = 8064


def _win_spans(windows):
    spans = []
    for wst, wlen in windows:
        lst = []
        pos = wst
        while pos < wst + wlen:
            seg = pos // SEG
            seg_end = min((seg + 1) * SEG, wst + wlen)
            kind = ('start' if pos == seg * SEG else 'finish')
            if seg_end == (seg + 1) * SEG:
                kind += '_done'
            lst.append((pos - wst, seg_end - pos, seg, kind))
            pos = seg_end
        spans.append((wst, wlen, lst))
    return spans


_SPANS_FULL = _win_spans(_WIN_FULL)
_SPANS_TAIL = _win_spans(_WIN_TAIL)


_UNROLL = 8


def _vmin_span(buf, par, r, bst, nv, acc0):
    nmain = nv // _UNROLL

    def vbody(t, acc, _bst=bst, _r=r, _par=par):
        base = _bst + t * (16 * _UNROLL)
        vs = [buf[_par, _r, pl.ds(base + u * 16, 16)] for u in range(_UNROLL)]
        m01, m23 = jnp.minimum(vs[0], vs[1]), jnp.minimum(vs[2], vs[3])
        m45, m67 = jnp.minimum(vs[4], vs[5]), jnp.minimum(vs[6], vs[7])
        m = jnp.minimum(jnp.minimum(m01, m23), jnp.minimum(m45, m67))
        return jnp.minimum(acc, m)

    acc = lax.fori_loop(0, nmain, vbody, acc0)
    for u in range(nmain * _UNROLL, nv):
        acc = jnp.minimum(acc, buf[par, r, pl.ds(bst + u * 16, 16)])
    return acc


def _sc_compute_window(buf, par, obuf, lst):
    for r in range(8):
        for (bst, ln, seg, kind) in lst:
            if kind.startswith('start'):
                acc0 = jnp.full((16,), jnp.inf, jnp.float32)
            else:
                acc0 = obuf[r, pl.ds(seg * 16, 16)]
            acc = _vmin_span(buf, par, r, bst, ln // 16, acc0)
            obuf[r, pl.ds(seg * 16, 16)] = acc


def _win_copy(e_hbm, g, lane_base, w, buf, sems, spans):
    wst, wlen, _ = spans[w]
    par = w % 2
    return pltpu.make_async_copy(
        e_hbm.at[pl.ds(g * 8, 8), pl.ds(lane_base + wst, wlen)],
        buf.at[par, :, pl.ds(0, wlen)], sems[par])


def _sc_segmin_kernel(e_hbm, m_hbm, buf, obuf, sem0, sem1):
    w = lax.axis_index("s") * 2 + lax.axis_index("c")
    g = w // 8
    k = w - g * 8
    sems = (sem0, sem1)
    nblk = jnp.where(k < 6, 8, 7)
    base_blk = jnp.where(k < 6, 8 * k, 48 + 7 * (k - 6))

    _win_copy(e_hbm, g, base_blk * 16000, 0, buf, sems, _SPANS_FULL).start()

    def blk_body(i, _):
        blk = base_blk + i

        @pl.when(i < nblk)
        def _():
            for wi in range(4):
                if wi < 3:
                    _win_copy(e_hbm, g, blk * 16000, wi + 1, buf, sems,
                              _SPANS_FULL).start()
                else:
                    @pl.when(i + 1 < nblk)
                    def _():
                        _win_copy(e_hbm, g, (blk + 1) * 16000, 0, buf, sems,
                                  _SPANS_FULL).start()
                _win_copy(e_hbm, g, blk * 16000, wi, buf, sems,
                          _SPANS_FULL).wait()
                _sc_compute_window(buf, wi % 2, obuf, _SPANS_FULL[wi][2])
            pltpu.sync_copy(obuf.at[:, pl.ds(0, 128)],
                            m_hbm.at[g, :, pl.ds(blk * 128, 128)])
        return 0

    lax.fori_loop(0, 8, blk_body, 0)

    @pl.when(k == 7)
    def _():
        for wi in range(2):
            wst, wlen, lst = _SPANS_TAIL[wi]
            pltpu.sync_copy(
                e_hbm.at[pl.ds(g * 8, 8), pl.ds(62 * 16000 + wst, wlen)],
                buf.at[wi % 2, :, pl.ds(0, wlen)])
            _sc_compute_window(buf, wi % 2, obuf, lst)
        pltpu.sync_copy(obuf.at[:, pl.ds(0, 128)],
                        m_hbm.at[g, :, pl.ds(62 * 128, 128)])


def _sc_segmin(x):
    B = x.shape[0]
    mesh = plsc.VectorSubcoreMesh(core_axis_name="c", subcore_axis_name="s")
    kfn = functools.partial(
        pl.kernel,
        mesh=mesh,
        out_type=jax.ShapeDtypeStruct((4, 8, MPAD), jnp.float32),
        scratch_types=[
            pltpu.VMEM((2, 8, 4096), jnp.float32),
            pltpu.VMEM((8, 128), jnp.float32),
            pltpu.SemaphoreType.DMA,
            pltpu.SemaphoreType.DMA,
        ],
    )(_sc_segmin_kernel)
    return kfn(x).reshape(B, MPAD)


def _selrows_kernel(m_ref, tail_ref, rl_ref):
    B = m_ref.shape[0]
    NS4 = MPAD // 16
    mm4 = jnp.min(m_ref[...].reshape(B, NS4, 16), axis=2)
    riota = jax.lax.broadcasted_iota(jnp.int32, (B, NS4), 1)
    jiota = jax.lax.broadcasted_iota(jnp.int32, (1, NSEL), 1)
    tmin = jnp.min(tail_ref[...], axis=1, keepdims=True)
    mm = jnp.where(riota >= NSEG, jnp.inf, mm4)
    mm = jnp.where(riota == NSEG - 1, jnp.minimum(mm, tmin), mm)

    def body(j, carry):
        mm, rl = carry
        v = jnp.min(mm, axis=1, keepdims=True)
        r = jnp.min(jnp.where(mm == v, riota, BIGI), axis=1, keepdims=True)
        rl = jnp.where(jiota == j, r, rl)
        mm = jnp.where(riota == r, jnp.inf, mm)
        return mm, rl

    rl0 = jnp.zeros((B, NSEL), jnp.int32)
    _, rl = jax.lax.fori_loop(0, NSEL, body, (mm, rl0))
    rl_ref[...] = rl


NSEL2 = 2 * NSEL
RW = 1000


def _gather_select_kernel(rl_smem, x_any, rlv_ref, ct_ref, tmask_ref, g_ref,
                          oi_ref, oe_ref, cand_ref, sem):
    B = rlv_ref.shape[0]

    def dma_start(t, _):
        b = t // NSEL2
        j = t - b * NSEL2
        rr = rl_smem[b, j]
        pltpu.make_async_copy(x_any.at[b, rr], cand_ref.at[b, j], sem).start()
        return 0

    jax.lax.fori_loop(0, B * NSEL2, dma_start, 0)

    def dma_wait(t, _):
        b = t // NSEL2
        j = t - b * NSEL2
        rr = rl_smem[b, j]
        pltpu.make_async_copy(x_any.at[b, rr], cand_ref.at[b, j], sem).wait()
        return 0

    jax.lax.fori_loop(0, B * NSEL2, dma_wait, 0)

    ciota3 = jax.lax.broadcasted_iota(jnp.int32, (B, NSEL2, RW), 2)
    rlv = rlv_ref[...]
    cond3 = (tmask_ref[...][:, :, None] == 1) & (ciota3 == ct_ref[...][:, None])
    cand_ref[...] = jnp.where(cond3, jnp.inf, cand_ref[...])

    cl = cand_ref[...]
    m1 = jnp.min(cl, axis=2)
    c1 = jnp.min(jnp.where(cl == m1[:, :, None], ciota3, BIGI), axis=2)
    t2 = jnp.where(ciota3 == c1[:, :, None], jnp.inf, cl)
    m2 = jnp.min(t2, axis=2)
    c2 = jnp.min(jnp.where(t2 == m2[:, :, None], ciota3, BIGI), axis=2)

    jiota = jax.lax.broadcasted_iota(jnp.int32, (B, NSEL2), 1)
    k30 = jax.lax.broadcasted_iota(jnp.int32, (1, K), 1)

    def clean(args):
        m1, c1, m2, c2, ti = args
        g3 = rlv[:, :, None] * RW + ciota3
        ex = jnp.zeros((B, NSEL2, RW), jnp.bool_)
        for s in range(K):
            ex = ex | (g3 == ti[:, s][:, None, None])
        cl = jnp.where(ex, jnp.inf, cand_ref[...])
        nm1 = jnp.min(cl, axis=2)
        nc1 = jnp.min(jnp.where(cl == nm1[:, :, None], ciota3, BIGI), axis=2)
        t2 = jnp.where(ciota3 == nc1[:, :, None], jnp.inf, cl)
        nm2 = jnp.min(t2, axis=2)
        nc2 = jnp.min(jnp.where(t2 == nm2[:, :, None], ciota3, BIGI), axis=2)
        return nm1, nc1, nm2, nc2, ti

    def body(k, carry):
        m1, c1, m2, c2, ti, te = carry
        stale = jnp.min(m1) == -jnp.inf
        m1, c1, m2, c2, ti = jax.lax.cond(stale, clean, lambda a: a,
                                          (m1, c1, m2, c2, ti))
        v = jnp.min(m1, axis=1, keepdims=True)
        rbest = jnp.min(jnp.where(m1 == v, rlv, BIGI), axis=1, keepdims=True)
        jstar = jnp.min(jnp.where((m1 == v) & (rlv == rbest), jiota, BIGI),
                        axis=1, keepdims=True)
        cstar = jnp.min(jnp.where(jiota == jstar, c1, BIGI), axis=1,
                        keepdims=True)
        ti = jnp.where(k30 == k, rbest * RW + cstar, ti)
        te = jnp.where(k30 == k, v, te)
        sel = jiota == jstar
        promo = jnp.where(m2 == jnp.inf, -jnp.inf, m2)
        m1 = jnp.where(sel, promo, m1)
        c1 = jnp.where(sel, c2, c1)
        m2 = jnp.where(sel, jnp.inf, m2)
        c2 = jnp.where(sel, BIGI, c2)
        return m1, c1, m2, c2, ti, te

    ti0 = jnp.full((B, K), BIGI, jnp.int32)
    te0 = jnp.zeros((B, K), jnp.float32)
    _, _, _, _, ti, te = jax.lax.fori_loop(0, K, body,
                                           (m1, c1, m2, c2, ti0, te0))

    g = g_ref[...]
    z = g - te[:, None, :]
    zmax = jnp.max(z, axis=2, keepdims=True)
    k30_3 = jax.lax.broadcasted_iota(jnp.int32, (B, N_NEG, K), 2)
    s = jnp.min(jnp.where(z == zmax, k30_3, BIGI), axis=2, keepdims=True)
    sel = k30_3 == s
    oi_ref[:, 0, :] = jnp.sum(jnp.where(sel, ti[:, None, :], 0), axis=2)
    oe_ref[:, 0, :] = jnp.sum(jnp.where(sel, te[:, None, :], 0.0), axis=2)


def kernel(energy, target):
    B, V = energy.shape
    xs = energy.reshape(B, V // 1000, 1000)
    t32 = target.astype(jnp.int32)
    rt = t32 // RW
    ct = t32 - rt * RW

    keys = jax.random.split(jax.random.key(1), B)
    gumbel = jax.vmap(lambda k: jax.random.gumbel(k, (N_NEG, K), jnp.float32))(keys)

    m16 = _sc_segmin(energy)

    tail64 = lax.slice(energy, (0, V - 64), (B, V))
    rl = pl.pallas_call(
        _selrows_kernel,
        in_specs=[pl.BlockSpec((B, MPAD), lambda: (0, 0)),
                  pl.BlockSpec((B, 64), lambda: (0, 0))],
        out_specs=pl.BlockSpec((B, NSEL), lambda: (0, 0)),
        out_shape=jax.ShapeDtypeStruct((B, NSEL), jnp.int32),
    )(m16, tail64)

    rl2 = jnp.concatenate([2 * rl, 2 * rl + 1], axis=1)
    tmask = (rl2 == rt[:, None]).astype(jnp.int32)

    grid_spec = pltpu.PrefetchScalarGridSpec(
        num_scalar_prefetch=1,
        grid=(1,),
        in_specs=[
            pl.BlockSpec(memory_space=pl.ANY),
            pl.BlockSpec((B, NSEL2), lambda i, rl_s: (0, 0)),
            pl.BlockSpec((B, 1), lambda i, rl_s: (0, 0)),
            pl.BlockSpec((B, NSEL2), lambda i, rl_s: (0, 0)),
            pl.BlockSpec((B, N_NEG, K), lambda i, rl_s: (0, 0, 0)),
        ],
        out_specs=[
            pl.BlockSpec((B, 1, N_NEG), lambda i, rl_s: (0, 0, 0)),
            pl.BlockSpec((B, 1, N_NEG), lambda i, rl_s: (0, 0, 0)),
        ],
        scratch_shapes=[
            pltpu.VMEM((B, NSEL2, RW), jnp.float32),
            pltpu.SemaphoreType.DMA,
        ],
    )
    oi, oe = pl.pallas_call(
        _gather_select_kernel,
        grid_spec=grid_spec,
        out_shape=[
            jax.ShapeDtypeStruct((B, 1, N_NEG), jnp.int32),
            jax.ShapeDtypeStruct((B, 1, N_NEG), jnp.float32),
        ],
    )(rl2, xs, rl2, ct[:, None], tmask, gumbel)
    return (oi, oe)


# speedup vs baseline: 1.5022x; 1.5022x over previous
import jax
import jax.numpy as jnp
from jax.experimental import pallas as pl
from jax.experimental.pallas import tpu as pltpu

N_NEG = 10
K = 30
NSEL = 32
BIGI = 2**30
R = 1000
C = 1000


def _rowmin_kernel(x_ref, m_ref):
    m_ref[0, 0, :] = jnp.min(x_ref[0], axis=1)


def _selrows_kernel(m_ref, rl_ref):
    mm = m_ref[:, 0, :]
    riota = jax.lax.broadcasted_iota(jnp.int32, mm.shape, 1)
    jiota = jax.lax.broadcasted_iota(jnp.int32, (1, NSEL), 1)

    def body(j, carry):
        mm, rl = carry
        v = jnp.min(mm, axis=1, keepdims=True)
        r = jnp.min(jnp.where(mm == v, riota, BIGI), axis=1, keepdims=True)
        rl = jnp.where(jiota == j, r, rl)
        mm = jnp.where(riota == r, jnp.inf, mm)
        return mm, rl

    B = mm.shape[0]
    rl0 = jnp.zeros((B, NSEL), jnp.int32)
    _, rl = jax.lax.fori_loop(0, NSEL, body, (mm, rl0))
    rl_ref[...] = rl


def _gather_select_kernel(rl_smem, x_any, rlv_ref, ct_ref, tmask_ref, g_ref,
                          oi_ref, oe_ref, cand_ref, sem):
    B = rlv_ref.shape[0]

    def dma_start(t, _):
        b = t // NSEL
        j = t - b * NSEL
        rr = rl_smem[b, j]
        pltpu.make_async_copy(x_any.at[b, rr], cand_ref.at[b, j], sem).start()
        return 0

    jax.lax.fori_loop(0, B * NSEL, dma_start, 0)

    def dma_wait(t, _):
        b = t // NSEL
        j = t - b * NSEL
        rr = rl_smem[b, j]
        pltpu.make_async_copy(x_any.at[b, rr], cand_ref.at[b, j], sem).wait()
        return 0

    jax.lax.fori_loop(0, B * NSEL, dma_wait, 0)

    ciota3 = jax.lax.broadcasted_iota(jnp.int32, (B, NSEL, C), 2)
    rlv = rlv_ref[...]
    cond3 = (tmask_ref[...][:, :, None] == 1) & (ciota3 == ct_ref[...][:, None])
    cand_ref[...] = jnp.where(cond3, jnp.inf, cand_ref[...])

    cl = cand_ref[...]
    m1 = jnp.min(cl, axis=2)
    c1 = jnp.min(jnp.where(cl == m1[:, :, None], ciota3, BIGI), axis=2)
    t2 = jnp.where(ciota3 == c1[:, :, None], jnp.inf, cl)
    m2 = jnp.min(t2, axis=2)
    c2 = jnp.min(jnp.where(t2 == m2[:, :, None], ciota3, BIGI), axis=2)

    jiota = jax.lax.broadcasted_iota(jnp.int32, (B, NSEL), 1)
    k30 = jax.lax.broadcasted_iota(jnp.int32, (1, K), 1)

    def clean(args):
        m1, c1, m2, c2, ti = args
        g3 = rlv[:, :, None] * C + ciota3
        ex = jnp.zeros((B, NSEL, C), jnp.bool_)
        for s in range(K):
            ex = ex | (g3 == ti[:, s][:, None, None])
        cl = jnp.where(ex, jnp.inf, cand_ref[...])
        nm1 = jnp.min(cl, axis=2)
        nc1 = jnp.min(jnp.where(cl == nm1[:, :, None], ciota3, BIGI), axis=2)
        t2 = jnp.where(ciota3 == nc1[:, :, None], jnp.inf, cl)
        nm2 = jnp.min(t2, axis=2)
        nc2 = jnp.min(jnp.where(t2 == nm2[:, :, None], ciota3, BIGI), axis=2)
        return nm1, nc1, nm2, nc2, ti

    def body(k, carry):
        m1, c1, m2, c2, ti, te = carry
        stale = jnp.min(m1) == -jnp.inf
        m1, c1, m2, c2, ti = jax.lax.cond(stale, clean, lambda a: a,
                                          (m1, c1, m2, c2, ti))
        v = jnp.min(m1, axis=1, keepdims=True)
        rbest = jnp.min(jnp.where(m1 == v, rlv, BIGI), axis=1, keepdims=True)
        jstar = jnp.min(jnp.where((m1 == v) & (rlv == rbest), jiota, BIGI),
                        axis=1, keepdims=True)
        cstar = jnp.min(jnp.where(jiota == jstar, c1, BIGI), axis=1,
                        keepdims=True)
        ti = jnp.where(k30 == k, rbest * C + cstar, ti)
        te = jnp.where(k30 == k, v, te)
        sel = jiota == jstar
        promo = jnp.where(m2 == jnp.inf, -jnp.inf, m2)
        m1 = jnp.where(sel, promo, m1)
        c1 = jnp.where(sel, c2, c1)
        m2 = jnp.where(sel, jnp.inf, m2)
        c2 = jnp.where(sel, BIGI, c2)
        return m1, c1, m2, c2, ti, te

    ti0 = jnp.full((B, K), BIGI, jnp.int32)
    te0 = jnp.zeros((B, K), jnp.float32)
    _, _, _, _, ti, te = jax.lax.fori_loop(0, K, body,
                                           (m1, c1, m2, c2, ti0, te0))

    g = g_ref[...]
    z = g - te[:, None, :]
    zmax = jnp.max(z, axis=2, keepdims=True)
    k30_3 = jax.lax.broadcasted_iota(jnp.int32, (B, N_NEG, K), 2)
    s = jnp.min(jnp.where(z == zmax, k30_3, BIGI), axis=2, keepdims=True)
    sel = k30_3 == s
    oi_ref[:, 0, :] = jnp.sum(jnp.where(sel, ti[:, None, :], 0), axis=2)
    oe_ref[:, 0, :] = jnp.sum(jnp.where(sel, te[:, None, :], 0.0), axis=2)


def kernel(energy, target):
    B, V = energy.shape
    x = energy.reshape(B, R, C)
    t32 = target.astype(jnp.int32)
    rt = t32 // C
    ct = t32 - rt * C

    keys = jax.random.split(jax.random.key(1), B)
    gumbel = jax.vmap(lambda k: jax.random.gumbel(k, (N_NEG, K), jnp.float32))(keys)

    m = pl.pallas_call(
        _rowmin_kernel,
        grid=(B,),
        in_specs=[pl.BlockSpec((1, R, C), lambda b: (b, 0, 0))],
        out_specs=pl.BlockSpec((1, 1, R), lambda b: (b, 0, 0)),
        out_shape=jax.ShapeDtypeStruct((B, 1, R), jnp.float32),
    )(x)

    rl = pl.pallas_call(
        _selrows_kernel,
        in_specs=[pl.BlockSpec((B, 1, R), lambda: (0, 0, 0))],
        out_specs=pl.BlockSpec((B, NSEL), lambda: (0, 0)),
        out_shape=jax.ShapeDtypeStruct((B, NSEL), jnp.int32),
    )(m)

    tmask = (rl == rt[:, None]).astype(jnp.int32)

    grid_spec = pltpu.PrefetchScalarGridSpec(
        num_scalar_prefetch=1,
        grid=(1,),
        in_specs=[
            pl.BlockSpec(memory_space=pl.ANY),
            pl.BlockSpec((B, NSEL), lambda i, rl_s: (0, 0)),
            pl.BlockSpec((B, 1), lambda i, rl_s: (0, 0)),
            pl.BlockSpec((B, NSEL), lambda i, rl_s: (0, 0)),
            pl.BlockSpec((B, N_NEG, K), lambda i, rl_s: (0, 0, 0)),
        ],
        out_specs=[
            pl.BlockSpec((B, 1, N_NEG), lambda i, rl_s: (0, 0, 0)),
            pl.BlockSpec((B, 1, N_NEG), lambda i, rl_s: (0, 0, 0)),
        ],
        scratch_shapes=[
            pltpu.VMEM((B, NSEL, C), jnp.float32),
            pltpu.SemaphoreType.DMA,
        ],
    )
    oi, oe = pl.pallas_call(
        _gather_select_kernel,
        grid_spec=grid_spec,
        out_shape=[
            jax.ShapeDtypeStruct((B, 1, N_NEG), jnp.int32),
            jax.ShapeDtypeStruct((B, 1, N_NEG), jnp.float32),
        ],
    )(rl, x, rl, ct[:, None], tmask, gumbel)
    return (oi, oe)
